# baseline (speedup 1.0000x reference)
import functools

import jax
import jax.numpy as jnp
from jax import lax
from jax.experimental import pallas as pl
from jax.experimental.pallas import tpu as pltpu
from jax.experimental.pallas import tpu_sc as plsc

_N = 10000
_NP = 10240
_E = 160000
_NSUB = 16
_CH = 100
_TPW = _E // _NSUB
_NCH = _TPW // _CH
_K = 1
_CPB = 20
_BPB = _CPB // (2 * _K)
_NBLK = _NCH // _CPB
_RPT = _NP // _NSUB
_CCH = 50
_CCPB = 50
_CNBLK = 2


def _sc_agg_body(h2n, srcx, dstx, agg_out, sidx_v, didx_v, *rest):
    rows_a = rest[0:_K]
    rows_b = rest[_K:2 * _K]
    agg_s, gsem_a, gsem_b, ssem_a, ssem_b = rest[2 * _K:]
    cid = lax.axis_index("c")
    sid = lax.axis_index("s")
    zvec = jnp.zeros((16,), jnp.float32)

    def _zero_rows(i, carry):
        for k in range(8):
            rows_a[0][i, pl.ds(16 * k, 16)] = zvec
        return carry

    lax.fori_loop(0, _CH, _zero_rows, 0)

    base_row = sid * _RPT
    zd = [pltpu.async_copy(rows_a[0].at[pl.ds(0, 40)],
                           agg_s.at[pl.ds(base_row + z * 40, 40)], ssem_a)
          for z in range(_RPT // 40)]
    for d in zd:
        d.wait()
    plsc.subcore_barrier()

    def _blk(blki, carry):
        fi = sid * _NBLK + blki
        pltpu.sync_copy(srcx.at[cid * (_NSUB * _NBLK) + fi], sidx_v)
        pltpu.sync_copy(dstx.at[fi], didx_v)
        for b in range(_K):
            pltpu.async_copy(h2n.at[sidx_v.at[b]], rows_a[b], gsem_a)

        def _body(t, carry2):
            r0 = t * 2 * _K
            gb = [pltpu.async_copy(h2n.at[sidx_v.at[r0 + _K + b]],
                                   rows_b[b], gsem_b)
                  for b in range(_K)]
            for b in range(_K):
                pltpu.make_async_copy(h2n.at[sidx_v.at[r0 + b]], rows_a[b],
                                      gsem_a).wait()
            sa = [pltpu.async_copy(rows_a[b], agg_s.at[didx_v.at[r0 + b]],
                                   ssem_a, add=True)
                  for b in range(_K)]
            for d in gb:
                d.wait()
            sb = [pltpu.async_copy(rows_b[b],
                                   agg_s.at[didx_v.at[r0 + _K + b]],
                                   ssem_b, add=True)
                  for b in range(_K)]
            for d in sa:
                d.wait()

            @pl.when(t < _BPB - 1)
            def _():
                for b in range(_K):
                    pltpu.async_copy(h2n.at[sidx_v.at[r0 + 2 * _K + b]],
                                     rows_a[b], gsem_a)

            for d in sb:
                d.wait()
            return carry2

        lax.fori_loop(0, _BPB, _body, 0)
        return carry

    lax.fori_loop(0, _NBLK, _blk, 0)
    plsc.subcore_barrier()

    pltpu.sync_copy(agg_s.at[pl.ds(base_row, _RPT)],
                    agg_out.at[pl.ds(cid * _NP + base_row, _RPT)])


@functools.cache
def _make_sc_agg():
    mesh = plsc.VectorSubcoreMesh(core_axis_name="c", subcore_axis_name="s")
    return pl.kernel(
        _sc_agg_body,
        out_type=jax.ShapeDtypeStruct((2 * _NP, 128), jnp.float32),
        mesh=mesh,
        scratch_types=[
            pltpu.VMEM((_CPB, _CH), jnp.int32),
            pltpu.VMEM((_CPB, _CH), jnp.int32),
            *[pltpu.VMEM((_CH, 128), jnp.float32) for _ in range(2 * _K)],
            pltpu.VMEM_SHARED((_NP, 128), jnp.float32),
            pltpu.SemaphoreType.DMA,
            pltpu.SemaphoreType.DMA,
            pltpu.SemaphoreType.DMA,
            pltpu.SemaphoreType.DMA,
        ],
    )


def _sc_cnt_body(dstc, cnta_out, cntb_out, didx_v, ones_v, cnt_s, ssem):
    cid = lax.axis_index("c")
    sid = lax.axis_index("s")
    zvec = jnp.zeros((16,), jnp.float32)
    ovec = jnp.ones((16,), jnp.float32)

    def _fill(vec):
        def _f(i, carry):
            for k in range(8):
                ones_v[i, pl.ds(16 * k, 16)] = vec
            return carry
        return _f

    lax.fori_loop(0, _CCH, _fill(zvec), 0)
    base_row = sid * _RPT
    zd = [pltpu.async_copy(ones_v.at[pl.ds(0, 40)],
                           cnt_s.at[pl.ds(base_row + z * 40, 40)], ssem)
          for z in range(_RPT // 40)]
    for d in zd:
        d.wait()
    lax.fori_loop(0, _CCH, _fill(ovec), 0)
    plsc.subcore_barrier()

    def _blk(blki, carry):
        fi = (cid * _NSUB + sid) * _CNBLK + blki
        pltpu.sync_copy(dstc.at[fi], didx_v)

        def _wave(w, carry2):
            sd = [pltpu.async_copy(ones_v, cnt_s.at[didx_v.at[w * 10 + j]],
                                   ssem, add=True)
                  for j in range(10)]
            for d in sd:
                d.wait()
            return carry2

        lax.fori_loop(0, _CCPB // 10, _wave, 0)
        return carry

    lax.fori_loop(0, _CNBLK, _blk, 0)
    plsc.subcore_barrier()

    @pl.when(cid == 0)
    def _():
        pltpu.sync_copy(cnt_s.at[pl.ds(base_row, _RPT)],
                        cnta_out.at[pl.ds(base_row, _RPT)])

    @pl.when(cid == 1)
    def _():
        pltpu.sync_copy(cnt_s.at[pl.ds(base_row, _RPT)],
                        cntb_out.at[pl.ds(base_row, _RPT)])


@functools.cache
def _make_sc_cnt():
    mesh = plsc.VectorSubcoreMesh(core_axis_name="c", subcore_axis_name="s")
    half = jax.ShapeDtypeStruct((_NP, 128), jnp.float32)
    return pl.kernel(
        _sc_cnt_body,
        out_type=(half, half),
        mesh=mesh,
        scratch_types=[
            pltpu.VMEM((_CCPB, _CCH), jnp.int32),
            pltpu.VMEM((_CCH, 128), jnp.float32),
            pltpu.VMEM_SHARED((_NP, 128), jnp.float32),
            pltpu.SemaphoreType.DMA,
        ],
    )


_BM = 1000


def _dot16(a, w):
    return jnp.dot(a.astype(jnp.bfloat16), w.astype(jnp.bfloat16),
                   preferred_element_type=jnp.float32)


def _tc_mlp2_body(h_ref, w1b, b1b, w2b, b2b, out_ref):
    h = jnp.concatenate([h_ref[0], h_ref[1]], axis=1)
    t2 = jnp.maximum(
        _dot16(h, w1b[...]) + b1b[...],
        0.0)
    t2 = _dot16(t2, w2b[...]) + b2b[...]
    out_ref[0] = t2[:, :128]
    out_ref[1] = t2[:, 128:]


def _tc_mlp2(h3, nn2):
    w1b, b1b, w2b, b2b = nn2
    full = lambda shape: pl.BlockSpec(shape, lambda i: (0,) * len(shape))
    blk3 = pl.BlockSpec((2, _BM, 128), lambda i: (0, i, 0))
    return pl.pallas_call(
        _tc_mlp2_body,
        grid=(_N // _BM,),
        in_specs=[
            blk3,
            full((256, 256)), full((1, 256)), full((256, 256)), full((1, 256)),
        ],
        out_specs=blk3,
        out_shape=jax.ShapeDtypeStruct((2, _NP, 128), jnp.float32),
    )(h3, w1b, b1b.reshape(1, 256), w2b, b2b.reshape(1, 256))


def _tc_mlp1add_body(agg_ref, t2_ref, cnta_ref, cntb_ref,
                     w1a, b1a, w2a, b2a, out_ref):
    agg = jnp.concatenate([agg_ref[0], agg_ref[1]], axis=1)
    cnt = cnta_ref[...][:, 0:1] + cntb_ref[...][:, 0:1]
    inv = 1.0 / jnp.maximum(cnt, 1.0)
    agg = agg * inv
    t1 = jnp.maximum(
        _dot16(agg, w1a[...]) + b1a[...],
        0.0)
    t1 = _dot16(t1, w2a[...]) + b2a[...]
    t = t1 + jnp.concatenate([t2_ref[0], t2_ref[1]], axis=1)
    out_ref[0] = t[:, :128]
    out_ref[1] = t[:, 128:]


def _tc_mlp1add(agg3, t2_3, cnta, cntb, nn1):
    w1a, b1a, w2a, b2a = nn1
    full = lambda shape: pl.BlockSpec(shape, lambda i: (0,) * len(shape))
    row = pl.BlockSpec((_BM, 128), lambda i: (i, 0))
    blk3 = pl.BlockSpec((2, _BM, 128), lambda i: (0, i, 0))
    return pl.pallas_call(
        _tc_mlp1add_body,
        grid=(_N // _BM,),
        in_specs=[
            blk3, blk3, row, row,
            full((256, 256)), full((1, 256)), full((256, 256)), full((1, 256)),
        ],
        out_specs=blk3,
        out_shape=jax.ShapeDtypeStruct((2, _NP, 128), jnp.float32),
    )(agg3, t2_3, cnta, cntb,
      w1a, b1a.reshape(1, 256), w2a, b2a.reshape(1, 256))


def _tc_final_body(h_ref, w1, b1, w2, b2, out_ref):
    h = jnp.concatenate([h_ref[0], h_ref[1]], axis=1)
    t = jnp.maximum(
        _dot16(h, w1[...]) + b1[...], 0.0)
    t = _dot16(t, w2[...]) + b2[...]
    out_ref[...] = 1.0 / (1.0 + jnp.exp(-t))


def _tc_final(h3, fparams):
    w1, b1, w2, b2 = fparams
    full = lambda shape: pl.BlockSpec(shape, lambda i: (0,) * len(shape))
    blk3 = pl.BlockSpec((2, _BM, 128), lambda i: (0, i, 0))
    return pl.pallas_call(
        _tc_final_body,
        grid=(_N // _BM,),
        in_specs=[
            blk3,
            full((256, 256)), full((1, 256)),
            full((256, 128)), full((1, 128)),
        ],
        out_specs=pl.BlockSpec((_BM, 128), lambda i: (i, 0)),
        out_shape=jax.ShapeDtypeStruct((_N, 128), jnp.float32),
    )(h3, w1, b1.reshape(1, 256), w2, b2.reshape(1, 128))


def kernel(x, edge_index, params):
    convs, final_p = params
    n = x.shape[0]
    src = edge_index[0]
    dst = edge_index[1]
    src2 = src.reshape(_NSUB * _NBLK, _CPB, _CH)
    srcx = jnp.concatenate([src2, src2 + _NP], axis=0)
    dstx = dst.reshape(_NSUB * _NBLK, _CPB, _CH)
    h2 = (jnp.zeros((2, _NP, 128), jnp.float32)
          .at[0, :n].set(x[:, :128])
          .at[1, :n].set(x[:, 128:])
          .reshape(2 * _NP, 128))
    dstc = dst.reshape(2 * _NSUB * _CNBLK, _CCPB, _CCH)
    cnta, cntb = _make_sc_cnt()(dstc)
    for nn1, nn2 in convs:
        t2 = _tc_mlp2(h2.reshape(2, _NP, 128), nn2)
        agg2 = _make_sc_agg()(h2, srcx, dstx)
        h3 = _tc_mlp1add(agg2.reshape(2, _NP, 128), t2, cnta, cntb, nn1)
        h2 = h3.reshape(2 * _NP, 128)
    return _tc_final(h2.reshape(2, _NP, 128), final_p)
